# baseline (speedup 1.0000x reference)
import functools

import jax
import jax.numpy as jnp
import numpy as np
from jax import lax
from jax.experimental import pallas as pl
from jax.experimental.pallas import tpu as pltpu
from jax.experimental.pallas import tpu_sc as plsc


def _hilbert_flat(n: int) -> np.ndarray:
    d = np.arange(n * n, dtype=np.int64)
    x = np.zeros_like(d)
    y = np.zeros_like(d)
    t = d.copy()
    s = 1
    while s < n:
        rx = 1 & (t // 2)
        ry = 1 & (t ^ rx)
        swap = ry == 0
        flip = swap & (rx == 1)
        xf = np.where(flip, s - 1 - x, x)
        yf = np.where(flip, s - 1 - y, y)
        xn = np.where(swap, yf, xf)
        yn = np.where(swap, xf, yf)
        x = xn + s * rx
        y = yn + s * ry
        t = t // 4
        s *= 2
    return x * n + y


@functools.cache
def _build(B, H, W, C):
    n_pix = H * W
    SQ = 16
    n_sq = n_pix // (SQ * SQ)
    n_items = B * n_sq

    info = plsc.get_sparse_core_info()
    NW = info.num_cores * info.num_subcores
    NC = info.num_cores
    per_w = n_items // NW
    sq_per_w = n_sq // (NW // B)

    mesh = plsc.VectorSubcoreMesh(core_axis_name="c", subcore_axis_name="s")

    @functools.partial(
        pl.kernel,
        mesh=mesh,
        out_type=jax.ShapeDtypeStruct((B, 1, n_pix, C), jnp.float32),
        compiler_params=pltpu.CompilerParams(use_tc_tiling_on_sc=False),
        scratch_types=[
            pltpu.VMEM((sq_per_w,), jnp.int32),
            pltpu.VMEM((sq_per_w,), jnp.int32),
            pltpu.VMEM((per_w * SQ * SQ // 2,), jnp.int32),
            pltpu.VMEM((SQ, SQ, C), jnp.float32),
            pltpu.VMEM((SQ, SQ, C), jnp.float32),
            pltpu.VMEM((SQ * SQ, C), jnp.float32),
            pltpu.SemaphoreType.DMA,
            pltpu.SemaphoreType.DMA,
            pltpu.SemaphoreType.DMA,
        ],
    )
    def gather_kernel(inp_hbm, xtab_hbm, ytab_hbm, lidx_hbm, out_hbm,
                      xw, yw, lidx_v, staged0, staged1, ob,
                      gsem0, gsem1, ssem):
        wid = lax.axis_index("s") * NC + lax.axis_index("c")
        b = wid // (NW // B)
        s0 = pl.multiple_of((wid % (NW // B)) * sq_per_w, sq_per_w)
        pltpu.sync_copy(xtab_hbm.at[pl.ds(pl.multiple_of(s0, 8), sq_per_w)], xw)
        pltpu.sync_copy(ytab_hbm.at[pl.ds(pl.multiple_of(s0, 8), sq_per_w)], yw)
        pltpu.sync_copy(
            lidx_hbm.at[pl.ds(pl.multiple_of(s0 * (SQ * SQ // 2), 64), per_w * SQ * SQ // 2)],
            lidx_v)

        staged = (staged0, staged1)
        gsem = (gsem0, gsem1)

        def fetch(xv, yv, lane, dst, sem):
            X = pl.multiple_of(xv[lane], SQ)
            Y = pl.multiple_of(yv[lane], SQ)
            pltpu.async_copy(
                inp_hbm.at[b, pl.ds(X, SQ), pl.ds(Y, SQ), :], dst, sem)

        fetch(xw[pl.ds(0, 16)], yw[pl.ds(0, 16)], 0, staged0, gsem0)

        def block_body(bl, _):
            xv = xw[pl.ds(pl.multiple_of(bl * 16, 16), 16)]
            yv = yw[pl.ds(pl.multiple_of(bl * 16, 16), 16)]
            for u16 in range(16):
                u = u16 % 2
                cu = staged[u]
                pltpu.make_async_copy(
                    inp_hbm.at[0, pl.ds(0, SQ), pl.ds(0, SQ), :],
                    cu, gsem[u]).wait()
                if u16 < 15:
                    fetch(xv, yv, u16 + 1, staged[1 - u], gsem[1 - u])
                else:
                    @pl.when(bl < per_w // 16 - 1)
                    def _():
                        nb = pl.multiple_of(bl * 16 + 16, 16)
                        fetch(xw[pl.ds(nb, 16)], yw[pl.ds(nb, 16)], 0,
                              staged[1 - u], gsem[1 - u])
                if u16 >= 1:
                    pltpu.make_async_copy(
                        ob, out_hbm.at[0, 0, pl.ds(0, SQ * SQ), :],
                        ssem).wait()
                else:
                    @pl.when(bl > 0)
                    def _():
                        pltpu.make_async_copy(
                            ob, out_hbm.at[0, 0, pl.ds(0, SQ * SQ), :],
                            ssem).wait()

                kbase = (bl * 16 + u16) * (SQ * SQ // 2)

                def group_body(g, _):
                    lvec = lidx_v[pl.ds(pl.multiple_of(kbase + g * 16, 16), 16)]
                    for r32 in range(32):
                        word = lvec[r32 // 2]
                        p = lax.bitwise_and(
                            lax.shift_right_logical(word, 16 * (r32 % 2)),
                            0xFFFF)
                        si = lax.shift_right_logical(p, 4)
                        sj = lax.bitwise_and(p, SQ - 1)
                        for c in range(C // 16):
                            ob[g * 32 + r32, pl.ds(c * 16, 16)] = (
                                cu[si, sj, pl.ds(c * 16, 16)])
                    return 0

                lax.fori_loop(0, SQ * SQ // 32, group_body, 0)

                d0 = (s0 + bl * 16 + u16) * SQ * SQ
                pltpu.async_copy(
                    ob, out_hbm.at[b, 0, pl.ds(pl.multiple_of(d0, SQ * SQ), SQ * SQ), :], ssem)
            return 0

        lax.fori_loop(0, per_w // 16, block_body, 0)

        pltpu.make_async_copy(
            ob, out_hbm.at[0, 0, pl.ds(0, SQ * SQ), :], ssem).wait()

    flat = _hilbert_flat(H)
    x = flat // W
    y = flat % W
    xs = x.reshape(n_sq, SQ * SQ)
    ys = y.reshape(n_sq, SQ * SQ)
    X = (xs.min(axis=1) // SQ) * SQ
    Y = (ys.min(axis=1) // SQ) * SQ
    lidx = (xs - X[:, None]) * SQ + (ys - Y[:, None])
    lidx_tab = np.ascontiguousarray(
        lidx.reshape(-1).astype(np.uint16)).view(np.int32)
    return (gather_kernel, jnp.asarray(X.astype(np.int32)),
            jnp.asarray(Y.astype(np.int32)), jnp.asarray(lidx_tab))


def kernel(inputs):
    B, H, W, C = inputs.shape
    gather_kernel, xtab, ytab, lidx_tab = _build(B, H, W, C)
    return gather_kernel(inputs, xtab, ytab, lidx_tab)


# speedup vs baseline: 1.2838x; 1.2838x over previous
import functools

import jax
import jax.numpy as jnp
import numpy as np
from jax import lax
from jax.experimental import pallas as pl
from jax.experimental.pallas import tpu as pltpu
from jax.experimental.pallas import tpu_sc as plsc


def _hilbert_flat(n: int) -> np.ndarray:
    d = np.arange(n * n, dtype=np.int64)
    x = np.zeros_like(d)
    y = np.zeros_like(d)
    t = d.copy()
    s = 1
    while s < n:
        rx = 1 & (t // 2)
        ry = 1 & (t ^ rx)
        swap = ry == 0
        flip = swap & (rx == 1)
        xf = np.where(flip, s - 1 - x, x)
        yf = np.where(flip, s - 1 - y, y)
        xn = np.where(swap, yf, xf)
        yn = np.where(swap, xf, yf)
        x = xn + s * rx
        y = yn + s * ry
        t = t // 4
        s *= 2
    return x * n + y


@functools.cache
def _build(B, H, W, C):
    n_pix = H * W
    SQ = 8
    n_sq = n_pix // (SQ * SQ)
    SQ_BITS = SQ.bit_length() - 1
    n_items = B * n_sq

    info = plsc.get_sparse_core_info()
    NW = info.num_cores * info.num_subcores
    NC = info.num_cores
    per_w = n_items // NW
    sq_per_w = n_sq // (NW // B)

    mesh = plsc.VectorSubcoreMesh(core_axis_name="c", subcore_axis_name="s")

    @functools.partial(
        pl.kernel,
        mesh=mesh,
        out_type=jax.ShapeDtypeStruct((B, 1, n_pix, C), jnp.float32),
        scratch_types=[
            pltpu.VMEM((sq_per_w,), jnp.int32),
            pltpu.VMEM((sq_per_w,), jnp.int32),
            pltpu.VMEM((per_w * SQ * SQ // 2,), jnp.int32),
            pltpu.VMEM((SQ, SQ, C), jnp.float32),
            pltpu.VMEM((SQ, SQ, C), jnp.float32),
            pltpu.VMEM((SQ * SQ, C), jnp.float32),
            pltpu.VMEM((SQ * SQ, C), jnp.float32),
            pltpu.SemaphoreType.DMA,
            pltpu.SemaphoreType.DMA,
            pltpu.SemaphoreType.DMA,
            pltpu.SemaphoreType.DMA,
        ],
    )
    def gather_kernel(inp_hbm, xtab_hbm, ytab_hbm, lidx_hbm, out_hbm,
                      xw, yw, lidx_v, staged0, staged1, outb0, outb1,
                      gsem0, gsem1, ssem0, ssem1):
        wid = lax.axis_index("s") * NC + lax.axis_index("c")
        b = wid // (NW // B)
        s0 = pl.multiple_of((wid % (NW // B)) * sq_per_w, sq_per_w)
        pltpu.sync_copy(xtab_hbm.at[pl.ds(pl.multiple_of(s0, 8), sq_per_w)], xw)
        pltpu.sync_copy(ytab_hbm.at[pl.ds(pl.multiple_of(s0, 8), sq_per_w)], yw)
        pltpu.sync_copy(
            lidx_hbm.at[pl.ds(pl.multiple_of(s0 * (SQ * SQ // 2), 64), per_w * SQ * SQ // 2)],
            lidx_v)

        staged = (staged0, staged1)
        outb = (outb0, outb1)
        gsem = (gsem0, gsem1)
        ssem = (ssem0, ssem1)

        def fetch(xv, yv, lane, dst, sem):
            X = pl.multiple_of(xv[lane], SQ)
            Y = pl.multiple_of(yv[lane], SQ)
            pltpu.async_copy(
                inp_hbm.at[b, pl.ds(X, SQ), pl.ds(Y, SQ), :], dst, sem)

        fetch(xw[pl.ds(0, 16)], yw[pl.ds(0, 16)], 0, staged0, gsem0)

        def block_body(bl, _):
            xv = xw[pl.ds(pl.multiple_of(bl * 16, 16), 16)]
            yv = yw[pl.ds(pl.multiple_of(bl * 16, 16), 16)]
            for u16 in range(16):
                u = u16 % 2
                cu, ob = staged[u], outb[u]
                pltpu.make_async_copy(
                    inp_hbm.at[0, pl.ds(0, SQ), pl.ds(0, SQ), :],
                    cu, gsem[u]).wait()
                if u16 < 15:
                    fetch(xv, yv, u16 + 1, staged[1 - u], gsem[1 - u])
                else:
                    @pl.when(bl < per_w // 16 - 1)
                    def _():
                        nb = pl.multiple_of(bl * 16 + 16, 16)
                        fetch(xw[pl.ds(nb, 16)], yw[pl.ds(nb, 16)], 0,
                              staged[1 - u], gsem[1 - u])
                if u16 >= 2:
                    pltpu.make_async_copy(
                        ob, out_hbm.at[0, 0, pl.ds(0, SQ * SQ), :],
                        ssem[u]).wait()
                else:
                    @pl.when(bl > 0)
                    def _():
                        pltpu.make_async_copy(
                            ob, out_hbm.at[0, 0, pl.ds(0, SQ * SQ), :],
                            ssem[u]).wait()

                kbase = (bl * 16 + u16) * (SQ * SQ // 2)

                def group_body(g, _):
                    lvec = lidx_v[pl.ds(pl.multiple_of(kbase + g * 16, 16), 16)]
                    for r32 in range(32):
                        word = lvec[r32 // 2]
                        p = lax.bitwise_and(
                            lax.shift_right_logical(word, 16 * (r32 % 2)),
                            0xFFFF)
                        si = lax.shift_right_logical(p, SQ_BITS)
                        sj = lax.bitwise_and(p, SQ - 1)
                        for c in range(C // 16):
                            ob[g * 32 + r32, pl.ds(c * 16, 16)] = (
                                cu[si, sj, pl.ds(c * 16, 16)])
                    return 0

                lax.fori_loop(0, SQ * SQ // 32, group_body, 0)

                d0 = (s0 + bl * 16 + u16) * SQ * SQ
                pltpu.async_copy(
                    ob, out_hbm.at[b, 0, pl.ds(pl.multiple_of(d0, SQ * SQ), SQ * SQ), :], ssem[u])
            return 0

        lax.fori_loop(0, per_w // 16, block_body, 0)

        for u in range(2):
            pltpu.make_async_copy(
                outb[u], out_hbm.at[0, 0, pl.ds(0, SQ * SQ), :],
                ssem[u]).wait()

    flat = _hilbert_flat(H)
    x = flat // W
    y = flat % W
    xs = x.reshape(n_sq, SQ * SQ)
    ys = y.reshape(n_sq, SQ * SQ)
    X = (xs.min(axis=1) // SQ) * SQ
    Y = (ys.min(axis=1) // SQ) * SQ
    lidx = (xs - X[:, None]) * SQ + (ys - Y[:, None])
    lidx_tab = np.ascontiguousarray(
        lidx.reshape(-1).astype(np.uint16)).view(np.int32)
    return (gather_kernel, jnp.asarray(X.astype(np.int32)),
            jnp.asarray(Y.astype(np.int32)), jnp.asarray(lidx_tab))


def kernel(inputs):
    B, H, W, C = inputs.shape
    gather_kernel, xtab, ytab, lidx_tab = _build(B, H, W, C)
    return gather_kernel(inputs, xtab, ytab, lidx_tab)


# speedup vs baseline: 1.2894x; 1.0043x over previous
import functools

import jax
import jax.numpy as jnp
import numpy as np
from jax import lax
from jax.experimental import pallas as pl
from jax.experimental.pallas import tpu as pltpu
from jax.experimental.pallas import tpu_sc as plsc


def _hilbert_flat(n: int) -> np.ndarray:
    d = np.arange(n * n, dtype=np.int64)
    x = np.zeros_like(d)
    y = np.zeros_like(d)
    t = d.copy()
    s = 1
    while s < n:
        rx = 1 & (t // 2)
        ry = 1 & (t ^ rx)
        swap = ry == 0
        flip = swap & (rx == 1)
        xf = np.where(flip, s - 1 - x, x)
        yf = np.where(flip, s - 1 - y, y)
        xn = np.where(swap, yf, xf)
        yn = np.where(swap, xf, yf)
        x = xn + s * rx
        y = yn + s * ry
        t = t // 4
        s *= 2
    return x * n + y


@functools.cache
def _build(B, H, W, C):
    n_pix = H * W
    SQ = 16
    n_sq = n_pix // (SQ * SQ)
    SQ_BITS = SQ.bit_length() - 1
    n_items = B * n_sq

    info = plsc.get_sparse_core_info()
    NW = info.num_cores * info.num_subcores
    NC = info.num_cores
    per_w = n_items // NW
    sq_per_w = n_sq // (NW // B)

    mesh = plsc.VectorSubcoreMesh(core_axis_name="c", subcore_axis_name="s")

    @functools.partial(
        pl.kernel,
        mesh=mesh,
        out_type=jax.ShapeDtypeStruct((B, 1, n_pix, C), jnp.float32),
        scratch_types=[
            pltpu.VMEM((sq_per_w,), jnp.int32),
            pltpu.VMEM((sq_per_w,), jnp.int32),
            pltpu.VMEM((per_w * SQ * SQ // 2,), jnp.int32),
            pltpu.VMEM((SQ * SQ, C), jnp.float32),
            pltpu.VMEM((SQ * SQ, C), jnp.float32),
            pltpu.VMEM((SQ * SQ, C), jnp.float32),
            pltpu.SemaphoreType.DMA,
            pltpu.SemaphoreType.DMA,
            pltpu.SemaphoreType.DMA,
        ],
    )
    def gather_kernel(inp_hbm, xtab_hbm, ytab_hbm, lidx_hbm, out_hbm,
                      xw, yw, lidx_v, staged0, staged1, ob,
                      gsem0, gsem1, ssem):
        wid = lax.axis_index("s") * NC + lax.axis_index("c")
        b = wid // (NW // B)
        s0 = pl.multiple_of((wid % (NW // B)) * sq_per_w, sq_per_w)
        pltpu.sync_copy(xtab_hbm.at[pl.ds(pl.multiple_of(s0, 8), sq_per_w)], xw)
        pltpu.sync_copy(ytab_hbm.at[pl.ds(pl.multiple_of(s0, 8), sq_per_w)], yw)
        pltpu.sync_copy(
            lidx_hbm.at[pl.ds(pl.multiple_of(s0 * (SQ * SQ // 2), 64), per_w * SQ * SQ // 2)],
            lidx_v)

        staged = (staged0, staged1)
        gsem = (gsem0, gsem1)

        def fetch(xv, yv, lane, dst, sem):
            X = xv[lane]
            Y = pl.multiple_of(yv[lane], SQ)

            def row_fetch(i, _):
                pltpu.async_copy(
                    inp_hbm.at[b, X + i, pl.ds(Y, SQ), :],
                    dst.at[pl.ds(pl.multiple_of(i * SQ, SQ), SQ), :], sem)
                return 0

            lax.fori_loop(0, SQ, row_fetch, 0)

        fetch(xw[pl.ds(0, 16)], yw[pl.ds(0, 16)], 0, staged0, gsem0)

        def block_body(bl, _):
            xv = xw[pl.ds(pl.multiple_of(bl * 16, 16), 16)]
            yv = yw[pl.ds(pl.multiple_of(bl * 16, 16), 16)]
            for u16 in range(16):
                u = u16 % 2
                cu = staged[u]
                pltpu.make_async_copy(
                    out_hbm.at[0, 0, pl.ds(0, SQ * SQ), :],
                    cu, gsem[u]).wait()
                if u16 < 15:
                    fetch(xv, yv, u16 + 1, staged[1 - u], gsem[1 - u])
                else:
                    @pl.when(bl < per_w // 16 - 1)
                    def _():
                        nb = pl.multiple_of(bl * 16 + 16, 16)
                        fetch(xw[pl.ds(nb, 16)], yw[pl.ds(nb, 16)], 0,
                              staged[1 - u], gsem[1 - u])
                if u16 >= 1:
                    pltpu.make_async_copy(
                        ob, out_hbm.at[0, 0, pl.ds(0, SQ * SQ), :],
                        ssem).wait()
                else:
                    @pl.when(bl > 0)
                    def _():
                        pltpu.make_async_copy(
                            ob, out_hbm.at[0, 0, pl.ds(0, SQ * SQ), :],
                            ssem).wait()

                kbase = (bl * 16 + u16) * (SQ * SQ // 2)

                def group_body(g, _):
                    lvec = lidx_v[pl.ds(pl.multiple_of(kbase + g * 16, 16), 16)]
                    for r32 in range(32):
                        word = lvec[r32 // 2]
                        p = lax.bitwise_and(
                            lax.shift_right_logical(word, 16 * (r32 % 2)),
                            0xFFFF)
                        for c in range(C // 16):
                            ob[g * 32 + r32, pl.ds(c * 16, 16)] = (
                                cu[p, pl.ds(c * 16, 16)])
                    return 0

                lax.fori_loop(0, SQ * SQ // 32, group_body, 0)

                d0 = (s0 + bl * 16 + u16) * SQ * SQ
                pltpu.async_copy(
                    ob, out_hbm.at[b, 0, pl.ds(pl.multiple_of(d0, SQ * SQ), SQ * SQ), :], ssem)
            return 0

        lax.fori_loop(0, per_w // 16, block_body, 0)

        pltpu.make_async_copy(
            ob, out_hbm.at[0, 0, pl.ds(0, SQ * SQ), :], ssem).wait()

    flat = _hilbert_flat(H)
    x = flat // W
    y = flat % W
    xs = x.reshape(n_sq, SQ * SQ)
    ys = y.reshape(n_sq, SQ * SQ)
    X = (xs.min(axis=1) // SQ) * SQ
    Y = (ys.min(axis=1) // SQ) * SQ
    lidx = (xs - X[:, None]) * SQ + (ys - Y[:, None])
    lidx_tab = np.ascontiguousarray(
        lidx.reshape(-1).astype(np.uint16)).view(np.int32)
    return (gather_kernel, jnp.asarray(X.astype(np.int32)),
            jnp.asarray(Y.astype(np.int32)), jnp.asarray(lidx_tab))


def kernel(inputs):
    B, H, W, C = inputs.shape
    gather_kernel, xtab, ytab, lidx_tab = _build(B, H, W, C)
    return gather_kernel(inputs, xtab, ytab, lidx_tab)


# speedup vs baseline: 1.2923x; 1.0023x over previous
import functools

import jax
import jax.numpy as jnp
import numpy as np
from jax import lax
from jax.experimental import pallas as pl
from jax.experimental.pallas import tpu as pltpu
from jax.experimental.pallas import tpu_sc as plsc


def _hilbert_flat(n: int) -> np.ndarray:
    d = np.arange(n * n, dtype=np.int64)
    x = np.zeros_like(d)
    y = np.zeros_like(d)
    t = d.copy()
    s = 1
    while s < n:
        rx = 1 & (t // 2)
        ry = 1 & (t ^ rx)
        swap = ry == 0
        flip = swap & (rx == 1)
        xf = np.where(flip, s - 1 - x, x)
        yf = np.where(flip, s - 1 - y, y)
        xn = np.where(swap, yf, xf)
        yn = np.where(swap, xf, yf)
        x = xn + s * rx
        y = yn + s * ry
        t = t // 4
        s *= 2
    return x * n + y


@functools.cache
def _build(B, H, W, C):
    n_pix = H * W
    SQ = 16
    n_sq = n_pix // (SQ * SQ)
    SQ_BITS = SQ.bit_length() - 1
    n_items = B * n_sq

    info = plsc.get_sparse_core_info()
    NW = info.num_cores * info.num_subcores
    NC = info.num_cores
    per_w = n_items // NW
    sq_per_w = n_sq // (NW // B)

    mesh = plsc.VectorSubcoreMesh(core_axis_name="c", subcore_axis_name="s")

    @functools.partial(
        pl.kernel,
        mesh=mesh,
        out_type=jax.ShapeDtypeStruct((B, 1, n_pix, C), jnp.float32),
        compiler_params=pltpu.CompilerParams(disable_bounds_checks=True),
        scratch_types=[
            pltpu.VMEM((sq_per_w,), jnp.int32),
            pltpu.VMEM((sq_per_w,), jnp.int32),
            pltpu.VMEM((per_w * SQ * SQ // 2,), jnp.int32),
            pltpu.VMEM((SQ * SQ, C), jnp.float32),
            pltpu.VMEM((SQ * SQ, C), jnp.float32),
            pltpu.VMEM((SQ * SQ, C), jnp.float32),
            pltpu.SemaphoreType.DMA,
            pltpu.SemaphoreType.DMA,
            pltpu.SemaphoreType.DMA,
        ],
    )
    def gather_kernel(inp_hbm, xtab_hbm, ytab_hbm, lidx_hbm, out_hbm,
                      xw, yw, lidx_v, staged0, staged1, ob,
                      gsem0, gsem1, ssem):
        wid = lax.axis_index("s") * NC + lax.axis_index("c")
        b = wid // (NW // B)
        s0 = pl.multiple_of((wid % (NW // B)) * sq_per_w, sq_per_w)
        pltpu.sync_copy(xtab_hbm.at[pl.ds(pl.multiple_of(s0, 8), sq_per_w)], xw)
        pltpu.sync_copy(ytab_hbm.at[pl.ds(pl.multiple_of(s0, 8), sq_per_w)], yw)
        pltpu.sync_copy(
            lidx_hbm.at[pl.ds(pl.multiple_of(s0 * (SQ * SQ // 2), 64), per_w * SQ * SQ // 2)],
            lidx_v)

        staged = (staged0, staged1)
        gsem = (gsem0, gsem1)

        def fetch(xv, yv, lane, dst, sem):
            X = xv[lane]
            Y = pl.multiple_of(yv[lane], SQ)

            def row_fetch(i, _):
                pltpu.async_copy(
                    inp_hbm.at[b, X + i, pl.ds(Y, SQ), :],
                    dst.at[pl.ds(pl.multiple_of(i * SQ, SQ), SQ), :], sem)
                return 0

            lax.fori_loop(0, SQ, row_fetch, 0)

        fetch(xw[pl.ds(0, 16)], yw[pl.ds(0, 16)], 0, staged0, gsem0)

        def block_body(bl, _):
            xv = xw[pl.ds(pl.multiple_of(bl * 16, 16), 16)]
            yv = yw[pl.ds(pl.multiple_of(bl * 16, 16), 16)]
            for u16 in range(16):
                u = u16 % 2
                cu = staged[u]
                pltpu.make_async_copy(
                    out_hbm.at[0, 0, pl.ds(0, SQ * SQ), :],
                    cu, gsem[u]).wait()
                if u16 < 15:
                    fetch(xv, yv, u16 + 1, staged[1 - u], gsem[1 - u])
                else:
                    @pl.when(bl < per_w // 16 - 1)
                    def _():
                        nb = pl.multiple_of(bl * 16 + 16, 16)
                        fetch(xw[pl.ds(nb, 16)], yw[pl.ds(nb, 16)], 0,
                              staged[1 - u], gsem[1 - u])
                if u16 >= 1:
                    pltpu.make_async_copy(
                        ob, out_hbm.at[0, 0, pl.ds(0, SQ * SQ), :],
                        ssem).wait()
                else:
                    @pl.when(bl > 0)
                    def _():
                        pltpu.make_async_copy(
                            ob, out_hbm.at[0, 0, pl.ds(0, SQ * SQ), :],
                            ssem).wait()

                kbase = (bl * 16 + u16) * (SQ * SQ // 2)

                def group_body(g, _):
                    lvec = lidx_v[pl.ds(pl.multiple_of(kbase + g * 16, 16), 16)]
                    for r32 in range(32):
                        word = lvec[r32 // 2]
                        p = lax.bitwise_and(
                            lax.shift_right_logical(word, 16 * (r32 % 2)),
                            0xFFFF)
                        for c in range(C // 16):
                            ob[g * 32 + r32, pl.ds(c * 16, 16)] = (
                                cu[p, pl.ds(c * 16, 16)])
                    return 0

                lax.fori_loop(0, SQ * SQ // 32, group_body, 0)

                d0 = (s0 + bl * 16 + u16) * SQ * SQ
                pltpu.async_copy(
                    ob, out_hbm.at[b, 0, pl.ds(pl.multiple_of(d0, SQ * SQ), SQ * SQ), :], ssem)
            return 0

        lax.fori_loop(0, per_w // 16, block_body, 0)

        pltpu.make_async_copy(
            ob, out_hbm.at[0, 0, pl.ds(0, SQ * SQ), :], ssem).wait()

    flat = _hilbert_flat(H)
    x = flat // W
    y = flat % W
    xs = x.reshape(n_sq, SQ * SQ)
    ys = y.reshape(n_sq, SQ * SQ)
    X = (xs.min(axis=1) // SQ) * SQ
    Y = (ys.min(axis=1) // SQ) * SQ
    lidx = (xs - X[:, None]) * SQ + (ys - Y[:, None])
    lidx_tab = np.ascontiguousarray(
        lidx.reshape(-1).astype(np.uint16)).view(np.int32)
    return (gather_kernel, jnp.asarray(X.astype(np.int32)),
            jnp.asarray(Y.astype(np.int32)), jnp.asarray(lidx_tab))


def kernel(inputs):
    B, H, W, C = inputs.shape
    gather_kernel, xtab, ytab, lidx_tab = _build(B, H, W, C)
    return gather_kernel(inputs, xtab, ytab, lidx_tab)


# speedup vs baseline: 1.6911x; 1.3085x over previous
import functools

import jax
import jax.numpy as jnp
import numpy as np
from jax import lax
from jax.experimental import pallas as pl
from jax.experimental.pallas import tpu as pltpu
from jax.experimental.pallas import tpu_sc as plsc


def _hilbert_flat(n: int) -> np.ndarray:
    d = np.arange(n * n, dtype=np.int64)
    x = np.zeros_like(d)
    y = np.zeros_like(d)
    t = d.copy()
    s = 1
    while s < n:
        rx = 1 & (t // 2)
        ry = 1 & (t ^ rx)
        swap = ry == 0
        flip = swap & (rx == 1)
        xf = np.where(flip, s - 1 - x, x)
        yf = np.where(flip, s - 1 - y, y)
        xn = np.where(swap, yf, xf)
        yn = np.where(swap, xf, yf)
        x = xn + s * rx
        y = yn + s * ry
        t = t // 4
        s *= 2
    return x * n + y


@functools.cache
def _build(B, H, W, C):
    n_pix = H * W
    SQ = 16
    n_sq = n_pix // (SQ * SQ)
    SQ_BITS = SQ.bit_length() - 1
    n_items = B * n_sq

    info = plsc.get_sparse_core_info()
    NW = info.num_cores * info.num_subcores
    NC = info.num_cores
    per_w = n_items // NW
    sq_per_w = n_sq // (NW // B)

    mesh = plsc.VectorSubcoreMesh(core_axis_name="c", subcore_axis_name="s")

    @functools.partial(
        pl.kernel,
        mesh=mesh,
        out_type=jax.ShapeDtypeStruct((B * n_pix, C), jnp.float32),
        compiler_params=pltpu.CompilerParams(disable_bounds_checks=True),
        scratch_types=[
            pltpu.VMEM((sq_per_w,), jnp.int32),
            pltpu.VMEM((sq_per_w,), jnp.int32),
            pltpu.VMEM((per_w * SQ * SQ // 2,), jnp.int32),
            pltpu.VMEM((SQ * SQ, C), jnp.float32),
            pltpu.VMEM((SQ * SQ, C), jnp.float32),
            pltpu.VMEM((SQ * SQ, C), jnp.float32),
            pltpu.SemaphoreType.DMA,
            pltpu.SemaphoreType.DMA,
            pltpu.SemaphoreType.DMA,
        ],
    )
    def gather_kernel(inp_hbm, xtab_hbm, ytab_hbm, lidx_hbm, out_hbm,
                      xw, yw, lidx_v, staged0, staged1, ob,
                      gsem0, gsem1, ssem):
        wid = lax.axis_index("s") * NC + lax.axis_index("c")
        b = wid // (NW // B)
        s0 = pl.multiple_of((wid % (NW // B)) * sq_per_w, sq_per_w)
        pltpu.sync_copy(xtab_hbm.at[pl.ds(pl.multiple_of(s0, 8), sq_per_w)], xw)
        pltpu.sync_copy(ytab_hbm.at[pl.ds(pl.multiple_of(s0, 8), sq_per_w)], yw)
        pltpu.sync_copy(
            lidx_hbm.at[pl.ds(pl.multiple_of(s0 * (SQ * SQ // 2), 64), per_w * SQ * SQ // 2)],
            lidx_v)

        staged = (staged0, staged1)
        gsem = (gsem0, gsem1)

        def fetch(xv, yv, lane, dst, sem):
            X = xv[lane]
            Y = pl.multiple_of(yv[lane], SQ)
            base = pl.multiple_of(b * n_pix + X * W + Y, SQ)

            def row_fetch(i, _):
                pltpu.async_copy(
                    inp_hbm.at[pl.ds(pl.multiple_of(base + i * W, SQ), SQ), :],
                    dst.at[pl.ds(pl.multiple_of(i * SQ, SQ), SQ), :], sem)
                return 0

            lax.fori_loop(0, SQ, row_fetch, 0)

        fetch(xw[pl.ds(0, 16)], yw[pl.ds(0, 16)], 0, staged0, gsem0)

        def block_body(bl, _):
            xv = xw[pl.ds(pl.multiple_of(bl * 16, 16), 16)]
            yv = yw[pl.ds(pl.multiple_of(bl * 16, 16), 16)]
            for u16 in range(16):
                u = u16 % 2
                cu = staged[u]
                pltpu.make_async_copy(
                    out_hbm.at[pl.ds(0, SQ * SQ), :],
                    cu, gsem[u]).wait()
                if u16 < 15:
                    fetch(xv, yv, u16 + 1, staged[1 - u], gsem[1 - u])
                else:
                    @pl.when(bl < per_w // 16 - 1)
                    def _():
                        nb = pl.multiple_of(bl * 16 + 16, 16)
                        fetch(xw[pl.ds(nb, 16)], yw[pl.ds(nb, 16)], 0,
                              staged[1 - u], gsem[1 - u])
                if u16 >= 1:
                    pltpu.make_async_copy(
                        ob, out_hbm.at[pl.ds(0, SQ * SQ), :],
                        ssem).wait()
                else:
                    @pl.when(bl > 0)
                    def _():
                        pltpu.make_async_copy(
                            ob, out_hbm.at[pl.ds(0, SQ * SQ), :],
                            ssem).wait()

                kbase = (bl * 16 + u16) * (SQ * SQ // 2)

                def group_body(g, _):
                    lvec = lidx_v[pl.ds(pl.multiple_of(kbase + g * 16, 16), 16)]
                    for r32 in range(32):
                        word = lvec[r32 // 2]
                        p = lax.bitwise_and(
                            lax.shift_right_logical(word, 16 * (r32 % 2)),
                            0xFFFF)
                        for c in range(C // 16):
                            ob[g * 32 + r32, pl.ds(c * 16, 16)] = (
                                cu[p, pl.ds(c * 16, 16)])
                    return 0

                lax.fori_loop(0, SQ * SQ // 32, group_body, 0)

                d0 = (b * n_pix // (SQ * SQ) + s0 + bl * 16 + u16) * SQ * SQ
                pltpu.async_copy(
                    ob, out_hbm.at[pl.ds(pl.multiple_of(d0, SQ * SQ), SQ * SQ), :], ssem)
            return 0

        lax.fori_loop(0, per_w // 16, block_body, 0)

        pltpu.make_async_copy(
            ob, out_hbm.at[pl.ds(0, SQ * SQ), :], ssem).wait()

    flat = _hilbert_flat(H)
    x = flat // W
    y = flat % W
    xs = x.reshape(n_sq, SQ * SQ)
    ys = y.reshape(n_sq, SQ * SQ)
    X = (xs.min(axis=1) // SQ) * SQ
    Y = (ys.min(axis=1) // SQ) * SQ
    lidx = (xs - X[:, None]) * SQ + (ys - Y[:, None])
    lidx_tab = np.ascontiguousarray(
        lidx.reshape(-1).astype(np.uint16)).view(np.int32)
    return (gather_kernel, jnp.asarray(X.astype(np.int32)),
            jnp.asarray(Y.astype(np.int32)), jnp.asarray(lidx_tab))


def kernel(inputs):
    B, H, W, C = inputs.shape
    gather_kernel, xtab, ytab, lidx_tab = _build(B, H, W, C)
    out = gather_kernel(inputs.reshape(B * H * W, C), xtab, ytab, lidx_tab)
    return out.reshape(B, 1, H * W, C)


# speedup vs baseline: 2.2652x; 1.3395x over previous
import functools

import jax
import jax.numpy as jnp
import numpy as np
from jax import lax
from jax.experimental import pallas as pl
from jax.experimental.pallas import tpu as pltpu
from jax.experimental.pallas import tpu_sc as plsc


def _hilbert_flat(n: int) -> np.ndarray:
    d = np.arange(n * n, dtype=np.int64)
    x = np.zeros_like(d)
    y = np.zeros_like(d)
    t = d.copy()
    s = 1
    while s < n:
        rx = 1 & (t // 2)
        ry = 1 & (t ^ rx)
        swap = ry == 0
        flip = swap & (rx == 1)
        xf = np.where(flip, s - 1 - x, x)
        yf = np.where(flip, s - 1 - y, y)
        xn = np.where(swap, yf, xf)
        yn = np.where(swap, xf, yf)
        x = xn + s * rx
        y = yn + s * ry
        t = t // 4
        s *= 2
    return x * n + y


@functools.cache
def _build(B, H, W, C):
    n_pix = H * W
    SQ = 16
    n_sq = n_pix // (SQ * SQ)
    SQ_BITS = SQ.bit_length() - 1
    n_items = B * n_sq

    info = plsc.get_sparse_core_info()
    NW = info.num_cores * info.num_subcores
    NC = info.num_cores
    per_w = n_items // NW
    sq_per_w = n_sq // (NW // B)

    mesh = plsc.VectorSubcoreMesh(core_axis_name="c", subcore_axis_name="s")

    @functools.partial(
        pl.kernel,
        mesh=mesh,
        out_type=jax.ShapeDtypeStruct((B * n_pix, C), jnp.float32),
        compiler_params=pltpu.CompilerParams(disable_bounds_checks=True),
        scratch_types=[
            pltpu.VMEM((sq_per_w,), jnp.int32),
            pltpu.VMEM((sq_per_w,), jnp.int32),
            pltpu.VMEM((per_w * SQ * SQ // 2,), jnp.int32),
            pltpu.VMEM((SQ * SQ, C), jnp.float32),
            pltpu.VMEM((SQ * SQ, C), jnp.float32),
            pltpu.VMEM((SQ * SQ, C), jnp.float32),
            pltpu.SemaphoreType.DMA,
            pltpu.SemaphoreType.DMA,
            pltpu.SemaphoreType.DMA,
        ],
    )
    def gather_kernel(inp_hbm, xtab_hbm, ytab_hbm, lidx_hbm, out_hbm,
                      xw, yw, lidx_v, staged0, staged1, ob,
                      gsem0, gsem1, ssem):
        wid = lax.axis_index("s") * NC + lax.axis_index("c")
        b = wid // (NW // B)
        s0 = pl.multiple_of((wid % (NW // B)) * sq_per_w, sq_per_w)
        pltpu.sync_copy(xtab_hbm.at[pl.ds(pl.multiple_of(s0, 8), sq_per_w)], xw)
        pltpu.sync_copy(ytab_hbm.at[pl.ds(pl.multiple_of(s0, 8), sq_per_w)], yw)
        pltpu.sync_copy(
            lidx_hbm.at[pl.ds(pl.multiple_of(s0 * (SQ * SQ // 2), 64), per_w * SQ * SQ // 2)],
            lidx_v)

        staged = (staged0, staged1)
        gsem = (gsem0, gsem1)

        def fetch(xv, yv, lane, dst, sem):
            X = xv[lane]
            Y = pl.multiple_of(yv[lane], SQ)
            base = pl.multiple_of(b * n_pix + X * W + Y, SQ)

            def row_fetch(i, _):
                pltpu.async_copy(
                    inp_hbm.at[pl.ds(pl.multiple_of(base + i * W, SQ), SQ), :],
                    dst.at[pl.ds(pl.multiple_of(i * SQ, SQ), SQ), :], sem)
                return 0

            lax.fori_loop(0, SQ, row_fetch, 0)

        fetch(xw[pl.ds(0, 16)], yw[pl.ds(0, 16)], 0, staged0, gsem0)

        def block_body(bl, _):
            xv = xw[pl.ds(pl.multiple_of(bl * 16, 16), 16)]
            yv = yw[pl.ds(pl.multiple_of(bl * 16, 16), 16)]
            for u16 in range(16):
                u = u16 % 2
                cu = staged[u]
                pltpu.make_async_copy(
                    out_hbm.at[pl.ds(0, SQ * SQ), :],
                    cu, gsem[u]).wait()
                if u16 < 15:
                    fetch(xv, yv, u16 + 1, staged[1 - u], gsem[1 - u])
                else:
                    @pl.when(bl < per_w // 16 - 1)
                    def _():
                        nb = pl.multiple_of(bl * 16 + 16, 16)
                        fetch(xw[pl.ds(nb, 16)], yw[pl.ds(nb, 16)], 0,
                              staged[1 - u], gsem[1 - u])
                if u16 >= 1:
                    pltpu.make_async_copy(
                        ob, out_hbm.at[pl.ds(0, SQ * SQ), :],
                        ssem).wait()
                else:
                    @pl.when(bl > 0)
                    def _():
                        pltpu.make_async_copy(
                            ob, out_hbm.at[pl.ds(0, SQ * SQ), :],
                            ssem).wait()

                kbase = (bl * 16 + u16) * (SQ * SQ // 2)

                def group_body(g, _):
                    lvec = lidx_v[pl.ds(pl.multiple_of(kbase + g * 16, 16), 16)]
                    for w4 in range(8):
                        vals = []
                        for r4 in range(4):
                            r32 = w4 * 4 + r4
                            word = lvec[r32 // 2]
                            pp = lax.bitwise_and(
                                lax.shift_right_logical(word, 16 * (r32 % 2)),
                                0xFFFF)
                            for c in range(C // 16):
                                vals.append(cu[pp, pl.ds(c * 16, 16)])
                        for r4 in range(4):
                            r32 = w4 * 4 + r4
                            for c in range(C // 16):
                                ob[g * 32 + r32, pl.ds(c * 16, 16)] = (
                                    vals[r4 * (C // 16) + c])
                    return 0

                lax.fori_loop(0, SQ * SQ // 32, group_body, 0)

                d0 = (b * n_pix // (SQ * SQ) + s0 + bl * 16 + u16) * SQ * SQ
                pltpu.async_copy(
                    ob, out_hbm.at[pl.ds(pl.multiple_of(d0, SQ * SQ), SQ * SQ), :], ssem)
            return 0

        lax.fori_loop(0, per_w // 16, block_body, 0)

        pltpu.make_async_copy(
            ob, out_hbm.at[pl.ds(0, SQ * SQ), :], ssem).wait()

    flat = _hilbert_flat(H)
    x = flat // W
    y = flat % W
    xs = x.reshape(n_sq, SQ * SQ)
    ys = y.reshape(n_sq, SQ * SQ)
    X = (xs.min(axis=1) // SQ) * SQ
    Y = (ys.min(axis=1) // SQ) * SQ
    lidx = (xs - X[:, None]) * SQ + (ys - Y[:, None])
    lidx_tab = np.ascontiguousarray(
        lidx.reshape(-1).astype(np.uint16)).view(np.int32)
    return (gather_kernel, jnp.asarray(X.astype(np.int32)),
            jnp.asarray(Y.astype(np.int32)), jnp.asarray(lidx_tab))


def kernel(inputs):
    B, H, W, C = inputs.shape
    gather_kernel, xtab, ytab, lidx_tab = _build(B, H, W, C)
    out = gather_kernel(inputs.reshape(B * H * W, C), xtab, ytab, lidx_tab)
    return out.reshape(B, 1, H * W, C)
